# baseline (speedup 1.0000x reference)
import functools

import jax
import jax.numpy as jnp
from jax import lax
from jax.experimental import pallas as pl
from jax.experimental.pallas import tpu as pltpu
from jax.experimental.pallas import tpu_sc as plsc

A = 16
MEM = 64
NC, NS = 2, 16
NW = NC * NS
LANES = 16
KBUF = 8
UNITS = 128


def _ngram_body(T, R, pc0_hbm, pc1_hbm, pc2_hbm, table2_hbm, table3_hbm,
                out_hbm, pc0_v, pc1_v, pc2_v, pidx_v, rows_v, *sems):
    sem_g = sems[:KBUF]
    sem_w = sems[KBUF:]
    wid = lax.axis_index("s") * NC + lax.axis_index("c")

    pltpu.sync_copy(pc0_hbm.at[pl.ds(wid * 8, 8)], pc0_v)
    pltpu.sync_copy(pc1_hbm.at[pl.ds(wid * 8, 8)], pc1_v)
    pltpu.sync_copy(pc2_hbm.at[pl.ds(wid * 8, 8)], pc2_v)

    iota = lax.iota(jnp.int32, LANES)
    p_vec = lax.bitwise_and(iota, 1)

    def idx_step(q, carry):
        for kc in range(8):
            for k_loc in range(8):
                k = kc * 8 + k_loc
                rvec = p_vec + 2 * k
                cur = pc0_v[q, k, pl.ds(0, LANES)]
                prev = pc1_v[q, k, pl.ds(0, LANES)]
                prev2 = pc2_v[q, k, pl.ds(0, LANES)]
                pidx_v[q * 16 + kc, pl.ds(k_loc * LANES, LANES)] = (
                    prev + cur * A + rvec * (A * A))
                pidx_v[q * 16 + 8 + kc, pl.ds(k_loc * LANES, LANES)] = (
                    prev2 + prev * A + cur * (A * A) + rvec * (A * A * A))
        return carry

    lax.fori_loop(0, 8, idx_step, 0)

    def wait_gathers(b):
        pltpu.make_async_copy(table2_hbm.at[pl.ds(0, 128)], rows_v.at[b],
                              sem_g[b]).wait()

    def wait_write(b):
        pltpu.make_async_copy(rows_v.at[b], out_hbm.at[pl.ds(0, 128)],
                              sem_w[b]).wait()

    zvec = jnp.zeros((LANES,), jnp.float32)

    def group_step(g, carry):
        for b in range(KBUF):
            u = g * KBUF + b

            @pl.when(g > 0)
            def _():
                wait_write(b)

            o = lax.rem(lax.div(u, 8), 2)

            @pl.when(o == 0)
            def _():
                pltpu.async_copy(table2_hbm.at[pidx_v.at[u]],
                                 rows_v.at[b], sem_g[b])

            @pl.when(o == 1)
            def _():
                pltpu.async_copy(table3_hbm.at[pidx_v.at[u]],
                                 rows_v.at[b], sem_g[b])

        for b in range(KBUF):
            u = g * KBUF + b
            wait_gathers(b)

            @pl.when(jnp.logical_and(wid == 0, u < 8))
            def _():
                for k_loc in range(8):
                    for i in range(2):
                        for c in range(MEM // LANES):
                            rows_v[b, k_loc * 16 + i,
                                   pl.ds(c * LANES, LANES)] = zvec

            @pl.when(jnp.logical_and(wid == 0,
                                     jnp.logical_and(u >= 8, u < 16)))
            def _():
                for k_loc in range(8):
                    for i in range(4):
                        for c in range(MEM // LANES):
                            rows_v[b, k_loc * 16 + i,
                                   pl.ds(c * LANES, LANES)] = zvec

            tr = wid * 8 + lax.div(u, 16)
            row0 = (tr * 128 + lax.rem(u, 16) * 8) * 16
            pltpu.async_copy(rows_v.at[b], out_hbm.at[pl.ds(row0, 128)],
                             sem_w[b])
        return carry

    lax.fori_loop(0, UNITS // KBUF, group_step, 0)
    for b in range(KBUF):
        wait_write(b)


@functools.partial(jax.jit, static_argnames=("T", "R"))
def _ngram_lookup(pc0, pc1, pc2, table2, table3, *, T, R):
    mesh = plsc.VectorSubcoreMesh(core_axis_name="c", subcore_axis_name="s")
    body = functools.partial(_ngram_body, T, R)
    return pl.kernel(
        body,
        out_type=jax.ShapeDtypeStruct((T * 2 * R, MEM), jnp.float32),
        mesh=mesh,
        compiler_params=pltpu.CompilerParams(use_tc_tiling_on_sc=False),
        scratch_types=[
            pltpu.VMEM((8, R // 2, LANES), jnp.int32),
            pltpu.VMEM((8, R // 2, LANES), jnp.int32),
            pltpu.VMEM((8, R // 2, LANES), jnp.int32),
            pltpu.VMEM((UNITS, R), jnp.int32),
            pltpu.VMEM((KBUF, R, MEM), jnp.float32),
        ] + [pltpu.SemaphoreType.DMA] * (2 * KBUF),
    )(pc0, pc1, pc2, table2, table3)


def kernel(route_codes_btr, table_ngram_2, table_ngram_3):
    B, S, R = route_codes_btr.shape
    codes = route_codes_btr.reshape(S, R).astype(jnp.int32)
    ext = jnp.zeros((S + 2, R), jnp.int32).at[2:].set(codes)

    def permute(arr):
        return (arr.reshape(S // 8, 8, R // 2, 2)
                .transpose(0, 2, 1, 3)
                .reshape(S // 8, R // 2, 16))

    pc0 = permute(ext[2:])
    pc1 = permute(ext[1:S + 1])
    pc2 = permute(ext[0:S])
    out = _ngram_lookup(pc0, pc1, pc2, table_ngram_2, table_ngram_3,
                        T=S, R=R)
    out = out.reshape(S // 8, 2 * R * MEM // 128, 8, 128)
    out = out.transpose(0, 2, 1, 3)
    return out.reshape(B, S, 2 * R * MEM)


# speedup vs baseline: 1.0033x; 1.0033x over previous
import functools

import jax
import jax.numpy as jnp
from jax import lax
from jax.experimental import pallas as pl
from jax.experimental.pallas import tpu as pltpu
from jax.experimental.pallas import tpu_sc as plsc

A = 16
MEM = 64
NC, NS = 2, 16
NW = NC * NS
LANES = 16
KBUF = 4
UNITS = 128


def _ngram_body(T, R, pc0_hbm, pc1_hbm, pc2_hbm, table2_hbm, table3_hbm,
                out_hbm, pc0_v, pc1_v, pc2_v, pidx_v, rows_v, *sems):
    sem_g = sems[:KBUF]
    sem_w = sems[KBUF:]
    wid = lax.axis_index("s") * NC + lax.axis_index("c")

    pltpu.sync_copy(pc0_hbm.at[pl.ds(wid * 8, 8)], pc0_v)
    pltpu.sync_copy(pc1_hbm.at[pl.ds(wid * 8, 8)], pc1_v)
    pltpu.sync_copy(pc2_hbm.at[pl.ds(wid * 8, 8)], pc2_v)

    iota = lax.iota(jnp.int32, LANES)
    p_vec = lax.bitwise_and(iota, 1)

    def idx_step(q, carry):
        for kc in range(8):
            for k_loc in range(8):
                k = kc * 8 + k_loc
                rvec = p_vec + 2 * k
                cur = pc0_v[q, k, pl.ds(0, LANES)]
                prev = pc1_v[q, k, pl.ds(0, LANES)]
                prev2 = pc2_v[q, k, pl.ds(0, LANES)]
                pidx_v[q * 16 + kc, pl.ds(k_loc * LANES, LANES)] = (
                    prev + cur * A + rvec * (A * A))
                pidx_v[q * 16 + 8 + kc, pl.ds(k_loc * LANES, LANES)] = (
                    prev2 + prev * A + cur * (A * A) + rvec * (A * A * A))
        return carry

    lax.fori_loop(0, 8, idx_step, 0)

    def wait_gathers(b):
        pltpu.make_async_copy(table2_hbm.at[pl.ds(0, 128)], rows_v.at[b],
                              sem_g[b]).wait()

    def wait_write(b):
        pltpu.make_async_copy(rows_v.at[b], out_hbm.at[pl.ds(0, 128)],
                              sem_w[b]).wait()

    zvec = jnp.zeros((LANES,), jnp.float32)

    def group_step(g, carry):
        for b in range(KBUF):
            u = g * KBUF + b

            @pl.when(g > 0)
            def _():
                wait_write(b)

            o = lax.rem(lax.div(u, 8), 2)

            @pl.when(o == 0)
            def _():
                pltpu.async_copy(table2_hbm.at[pidx_v.at[u]],
                                 rows_v.at[b], sem_g[b])

            @pl.when(o == 1)
            def _():
                pltpu.async_copy(table3_hbm.at[pidx_v.at[u]],
                                 rows_v.at[b], sem_g[b])

        for b in range(KBUF):
            u = g * KBUF + b
            wait_gathers(b)

            @pl.when(jnp.logical_and(wid == 0, u < 8))
            def _():
                for k_loc in range(8):
                    for i in range(2):
                        for c in range(MEM // LANES):
                            rows_v[b, k_loc * 16 + i,
                                   pl.ds(c * LANES, LANES)] = zvec

            @pl.when(jnp.logical_and(wid == 0,
                                     jnp.logical_and(u >= 8, u < 16)))
            def _():
                for k_loc in range(8):
                    for i in range(4):
                        for c in range(MEM // LANES):
                            rows_v[b, k_loc * 16 + i,
                                   pl.ds(c * LANES, LANES)] = zvec

            tr = wid * 8 + lax.div(u, 16)
            row0 = (tr * 128 + lax.rem(u, 16) * 8) * 16
            pltpu.async_copy(rows_v.at[b], out_hbm.at[pl.ds(row0, 128)],
                             sem_w[b])
        return carry

    lax.fori_loop(0, UNITS // KBUF, group_step, 0)
    for b in range(KBUF):
        wait_write(b)


@functools.partial(jax.jit, static_argnames=("T", "R"))
def _ngram_lookup(pc0, pc1, pc2, table2, table3, *, T, R):
    mesh = plsc.VectorSubcoreMesh(core_axis_name="c", subcore_axis_name="s")
    body = functools.partial(_ngram_body, T, R)
    return pl.kernel(
        body,
        out_type=jax.ShapeDtypeStruct((T * 2 * R, MEM), jnp.float32),
        mesh=mesh,
        compiler_params=pltpu.CompilerParams(use_tc_tiling_on_sc=False),
        scratch_types=[
            pltpu.VMEM((8, R // 2, LANES), jnp.int32),
            pltpu.VMEM((8, R // 2, LANES), jnp.int32),
            pltpu.VMEM((8, R // 2, LANES), jnp.int32),
            pltpu.VMEM((UNITS, R), jnp.int32),
            pltpu.VMEM((KBUF, R, MEM), jnp.float32),
        ] + [pltpu.SemaphoreType.DMA] * (2 * KBUF),
    )(pc0, pc1, pc2, table2, table3)


def kernel(route_codes_btr, table_ngram_2, table_ngram_3):
    B, S, R = route_codes_btr.shape
    codes = route_codes_btr.reshape(S, R).astype(jnp.int32)
    ext = jnp.zeros((S + 2, R), jnp.int32).at[2:].set(codes)

    def permute(arr):
        return (arr.reshape(S // 8, 8, R // 2, 2)
                .transpose(0, 2, 1, 3)
                .reshape(S // 8, R // 2, 16))

    pc0 = permute(ext[2:])
    pc1 = permute(ext[1:S + 1])
    pc2 = permute(ext[0:S])
    out = _ngram_lookup(pc0, pc1, pc2, table_ngram_2, table_ngram_3,
                        T=S, R=R)
    out = out.reshape(S // 8, 2 * R * MEM // 128, 8, 128)
    out = out.transpose(0, 2, 1, 3)
    return out.reshape(B, S, 2 * R * MEM)


# speedup vs baseline: 1.0037x; 1.0004x over previous
import functools

import jax
import jax.numpy as jnp
from jax import lax
from jax.experimental import pallas as pl
from jax.experimental.pallas import tpu as pltpu
from jax.experimental.pallas import tpu_sc as plsc

A = 16
MEM = 64
NC, NS = 2, 16
NW = NC * NS
LANES = 16
KBUF = 4
UNITS = 128


def _ngram_body(T, R, pc0_hbm, pc1_hbm, pc2_hbm, table2_hbm, table3_hbm,
                out_hbm, pc0_v, pc1_v, pc2_v, pidx_v, rows_v, *sems):
    sem_g = sems[:KBUF]
    sem_w = sems[KBUF:]
    wid = lax.axis_index("s") * NC + lax.axis_index("c")

    pltpu.sync_copy(pc0_hbm.at[pl.ds(wid * 8, 8)], pc0_v)
    pltpu.sync_copy(pc1_hbm.at[pl.ds(wid * 8, 8)], pc1_v)
    pltpu.sync_copy(pc2_hbm.at[pl.ds(wid * 8, 8)], pc2_v)

    iota = lax.iota(jnp.int32, LANES)
    p_vec = lax.bitwise_and(iota, 1)

    def idx_step(q, carry):
        for kc in range(8):
            for k_loc in range(8):
                k = kc * 8 + k_loc
                rvec = p_vec + 2 * k
                cur = pc0_v[q, k, pl.ds(0, LANES)]
                prev = pc1_v[q, k, pl.ds(0, LANES)]
                prev2 = pc2_v[q, k, pl.ds(0, LANES)]
                pidx_v[q * 16 + kc, pl.ds(k_loc * LANES, LANES)] = (
                    prev + cur * A + rvec * (A * A))
                pidx_v[q * 16 + 8 + kc, pl.ds(k_loc * LANES, LANES)] = (
                    prev2 + prev * A + cur * (A * A) + rvec * (A * A * A))
        return carry

    lax.fori_loop(0, 8, idx_step, 0)

    def wait_gathers(b):
        pltpu.make_async_copy(table2_hbm.at[pl.ds(0, 128)], rows_v.at[b],
                              sem_g[b]).wait()

    def wait_write(b):
        pltpu.make_async_copy(rows_v.at[b], out_hbm.at[pl.ds(0, 128)],
                              sem_w[b]).wait()

    zvec = jnp.zeros((LANES,), jnp.float32)

    def group_step(g, carry):
        for b in range(KBUF):
            v = g * KBUF + b
            u = (lax.div(v, 16) * 16 + lax.rem(v, 2) * 8
                 + lax.div(lax.rem(v, 16), 2))

            @pl.when(g > 0)
            def _():
                wait_write(b)

            o = lax.rem(lax.div(u, 8), 2)

            @pl.when(o == 0)
            def _():
                pltpu.async_copy(table2_hbm.at[pidx_v.at[u]],
                                 rows_v.at[b], sem_g[b])

            @pl.when(o == 1)
            def _():
                pltpu.async_copy(table3_hbm.at[pidx_v.at[u]],
                                 rows_v.at[b], sem_g[b])

        for b in range(KBUF):
            v = g * KBUF + b
            u = (lax.div(v, 16) * 16 + lax.rem(v, 2) * 8
                 + lax.div(lax.rem(v, 16), 2))
            wait_gathers(b)

            @pl.when(jnp.logical_and(wid == 0, u < 8))
            def _():
                for k_loc in range(8):
                    for i in range(2):
                        for c in range(MEM // LANES):
                            rows_v[b, k_loc * 16 + i,
                                   pl.ds(c * LANES, LANES)] = zvec

            @pl.when(jnp.logical_and(wid == 0,
                                     jnp.logical_and(u >= 8, u < 16)))
            def _():
                for k_loc in range(8):
                    for i in range(4):
                        for c in range(MEM // LANES):
                            rows_v[b, k_loc * 16 + i,
                                   pl.ds(c * LANES, LANES)] = zvec

            tr = wid * 8 + lax.div(u, 16)
            row0 = (tr * 128 + lax.rem(u, 16) * 8) * 16
            pltpu.async_copy(rows_v.at[b], out_hbm.at[pl.ds(row0, 128)],
                             sem_w[b])
        return carry

    lax.fori_loop(0, UNITS // KBUF, group_step, 0)
    for b in range(KBUF):
        wait_write(b)


@functools.partial(jax.jit, static_argnames=("T", "R"))
def _ngram_lookup(pc0, pc1, pc2, table2, table3, *, T, R):
    mesh = plsc.VectorSubcoreMesh(core_axis_name="c", subcore_axis_name="s")
    body = functools.partial(_ngram_body, T, R)
    return pl.kernel(
        body,
        out_type=jax.ShapeDtypeStruct((T * 2 * R, MEM), jnp.float32),
        mesh=mesh,
        compiler_params=pltpu.CompilerParams(use_tc_tiling_on_sc=False),
        scratch_types=[
            pltpu.VMEM((8, R // 2, LANES), jnp.int32),
            pltpu.VMEM((8, R // 2, LANES), jnp.int32),
            pltpu.VMEM((8, R // 2, LANES), jnp.int32),
            pltpu.VMEM((UNITS, R), jnp.int32),
            pltpu.VMEM((KBUF, R, MEM), jnp.float32),
        ] + [pltpu.SemaphoreType.DMA] * (2 * KBUF),
    )(pc0, pc1, pc2, table2, table3)


def kernel(route_codes_btr, table_ngram_2, table_ngram_3):
    B, S, R = route_codes_btr.shape
    codes = route_codes_btr.reshape(S, R).astype(jnp.int32)
    ext = jnp.zeros((S + 2, R), jnp.int32).at[2:].set(codes)

    def permute(arr):
        return (arr.reshape(S // 8, 8, R // 2, 2)
                .transpose(0, 2, 1, 3)
                .reshape(S // 8, R // 2, 16))

    pc0 = permute(ext[2:])
    pc1 = permute(ext[1:S + 1])
    pc2 = permute(ext[0:S])
    out = _ngram_lookup(pc0, pc1, pc2, table_ngram_2, table_ngram_3,
                        T=S, R=R)
    out = out.reshape(S // 8, 2 * R * MEM // 128, 8, 128)
    out = out.transpose(0, 2, 1, 3)
    return out.reshape(B, S, 2 * R * MEM)
